# baseline (speedup 1.0000x reference)
import functools

import jax
import jax.numpy as jnp
from jax import lax
from jax.experimental import pallas as pl
from jax.experimental.pallas import tpu as pltpu
from jax.experimental.pallas import tpu_sc as plsc

E = 8
K = 2
D = 1024
F = 2048
T = 2048
A = T * K
B = 128
NB = A // B + E
S = NB * B

NC = 2
NS = 16
NW = NC * NS


def _router_body(g_ref, e1_ref, e2_ref, w1_ref, w2_ref, cnt_ref):
    s = g_ref[...]
    ii = lax.broadcasted_iota(jnp.int32, (T, E), 1)
    m1 = jnp.max(s, axis=1, keepdims=True)
    i1 = jnp.min(jnp.where(s == m1, ii, E), axis=1, keepdims=True)
    s2 = jnp.where(ii == i1, -jnp.inf, s)
    m2 = jnp.max(s2, axis=1, keepdims=True)
    i2 = jnp.min(jnp.where(s2 == m2, ii, E), axis=1, keepdims=True)
    e1_ref[...] = i1
    e2_ref[...] = i2
    w1_ref[...] = jax.nn.sigmoid(m1 - m2)
    w2_ref[...] = jax.nn.sigmoid(m2 - m1)
    sel = (ii == i1).astype(jnp.int32) + (ii == i2).astype(jnp.int32)
    cnt_ref[...] = jnp.sum(sel, axis=0, keepdims=True)


def _router(gating):
    return pl.pallas_call(
        _router_body,
        out_shape=(
            jax.ShapeDtypeStruct((T, 1), jnp.int32),
            jax.ShapeDtypeStruct((T, 1), jnp.int32),
            jax.ShapeDtypeStruct((T, 1), jnp.float32),
            jax.ShapeDtypeStruct((T, 1), jnp.float32),
            jax.ShapeDtypeStruct((1, E), jnp.int32),
        ),
    )(gating)


def _routing_metadata(e1, e2, cnt):
    ea = jnp.concatenate([e1.reshape(T), e2.reshape(T)])
    ii = jnp.arange(A, dtype=jnp.int32)
    _, order = lax.sort((ea, ii), num_keys=1)
    tok_sorted = order % T
    g = cnt.reshape(E)
    c = (g + B - 1) // B
    cumg = jnp.cumsum(g).astype(jnp.int32)
    padc = (c * B - g).astype(jnp.int32)
    pad = jnp.sum(jnp.where(ii[:, None] >= cumg[None, :], padc[None, :], 0),
                  axis=1).astype(jnp.int32)
    slot_sorted = ii + pad
    _, inv_slot = lax.sort((order, slot_sorted), num_keys=1)
    p0 = inv_slot[:T]
    p1 = inv_slot[T:]
    block_expert = jnp.repeat(
        jnp.arange(E, dtype=jnp.int32), c, total_repeat_length=NB)
    nb_real = jnp.sum(c).astype(jnp.int32)
    bidx = jnp.arange(NB, dtype=jnp.int32)
    valid = (bidx < nb_real).astype(jnp.int32)
    chg_raw = jnp.concatenate(
        [jnp.ones(1, jnp.bool_), block_expert[1:] != block_expert[:-1]])
    chg = (chg_raw & (valid == 1)).astype(jnp.int32)
    cpos = jnp.where(chg == 1, bidx, NB - 1)
    nxtpos_incl = jnp.flip(lax.cummin(jnp.flip(cpos)))
    nxtpos = jnp.concatenate([nxtpos_incl[1:], jnp.full(1, NB - 1, jnp.int32)])
    nxt = block_expert[nxtpos]
    return tok_sorted, slot_sorted, p0, p1, block_expert, valid, chg, nxt


_RPW = A // NW
_CH = 32
_DROUNDS = _RPW // _CH


def _dispatch(x, tok_sorted, slot_sorted):
    mesh = plsc.VectorSubcoreMesh(
        core_axis_name="c", subcore_axis_name="s", num_cores=NC, num_subcores=NS)

    @functools.partial(
        pl.kernel,
        out_type=jax.ShapeDtypeStruct((S, D), jnp.float32),
        mesh=mesh,
        scratch_types=[
            [pltpu.VMEM((_CH,), jnp.int32)] * 2,
            [pltpu.VMEM((_CH,), jnp.int32)] * 2,
            [pltpu.VMEM((_CH, D), jnp.float32)] * 2,
            [pltpu.SemaphoreType.DMA] * 2,
            [pltpu.SemaphoreType.DMA] * 2,
        ],
    )
    def k(x_hbm, tok_hbm, slot_hbm, out_hbm, toks, slots, bufs, gsems, ssems):
        wid = lax.axis_index("s") * NC + lax.axis_index("c")
        base = wid * _RPW
        gcp = [None] * _DROUNDS
        scp = [None] * _DROUNDS

        def fire(r):
            p = r % 2
            pltpu.sync_copy(tok_hbm.at[pl.ds(base + r * _CH, _CH)], toks[p])
            pltpu.sync_copy(slot_hbm.at[pl.ds(base + r * _CH, _CH)], slots[p])
            gcp[r] = pltpu.async_copy(x_hbm.at[toks[p]], bufs[p], gsems[p])

        fire(0)
        for r in range(_DROUNDS):
            p = r % 2
            if r + 1 < _DROUNDS:
                fire(r + 1)
            gcp[r].wait()
            scp[r] = pltpu.async_copy(bufs[p], out_hbm.at[slots[p]], ssems[p])
            if r >= 1:
                scp[r - 1].wait()
        scp[_DROUNDS - 1].wait()

    return k(x, tok_sorted, slot_sorted)


def _ffn_body(be_ref, va_ref, chg_ref, nxt_ref, xs_ref, gup_ref, down_ref,
              ys_ref, gup_land, down_land, gup16, down16, gsem, dsem):
    b = pl.program_id(0)
    live = va_ref[b] != 0

    @pl.when(b == 0)
    def _():
        e0 = be_ref[0]
        pltpu.make_async_copy(gup_ref.at[e0], gup_land, gsem).start()
        pltpu.make_async_copy(down_ref.at[e0], down_land, dsem).start()

    @pl.when(chg_ref[b] == 1)
    def _():
        pltpu.make_async_copy(gup_ref.at[0], gup_land, gsem).wait()
        pltpu.make_async_copy(down_ref.at[0], down_land, dsem).wait()
        gup16[...] = gup_land[...].astype(jnp.bfloat16)
        down16[...] = down_land[...].astype(jnp.bfloat16)
        nxt = nxt_ref[b]

        @pl.when(nxt != be_ref[b])
        def _():
            pltpu.make_async_copy(gup_ref.at[nxt], gup_land, gsem).start()
            pltpu.make_async_copy(down_ref.at[nxt], down_land, dsem).start()

    @pl.when(live)
    def _():
        xb = xs_ref[...].astype(jnp.bfloat16)
        acc = lax.dot_general(xb, gup16[...], (((1,), (1,)), ((), ())),
                              preferred_element_type=jnp.float32)
        gte = acc[:, :F]
        up = acc[:, F:]
        h = (gte * jax.nn.sigmoid(gte) * up).astype(jnp.bfloat16)
        y = lax.dot_general(h, down16[...], (((1,), (1,)), ((), ())),
                            preferred_element_type=jnp.float32)
        ys_ref[...] = y


def _ffn(xs, gup, down, block_expert, valid, chg, nxt):
    grid_spec = pltpu.PrefetchScalarGridSpec(
        num_scalar_prefetch=4,
        grid=(NB,),
        in_specs=[
            pl.BlockSpec((B, D), lambda b, be, va, ch, nx: (b, 0)),
            pl.BlockSpec(memory_space=pltpu.MemorySpace.HBM),
            pl.BlockSpec(memory_space=pltpu.MemorySpace.HBM),
        ],
        out_specs=pl.BlockSpec((B, D), lambda b, be, va, ch, nx: (b, 0)),
        scratch_shapes=[
            pltpu.VMEM((2 * F, D), jnp.float32),
            pltpu.VMEM((D, F), jnp.float32),
            pltpu.VMEM((2 * F, D), jnp.bfloat16),
            pltpu.VMEM((D, F), jnp.bfloat16),
            pltpu.SemaphoreType.DMA,
            pltpu.SemaphoreType.DMA,
        ],
    )
    return pl.pallas_call(
        _ffn_body,
        grid_spec=grid_spec,
        out_shape=jax.ShapeDtypeStruct((S, D), jnp.float32),
    )(block_expert, valid, chg, nxt, xs, gup, down)


_PER_T = T // NW
_CHT = 16
_TROUNDS = _PER_T // _CHT


def _combine(ys, p0, p1, w0, w1):
    mesh = plsc.VectorSubcoreMesh(
        core_axis_name="c", subcore_axis_name="s", num_cores=NC, num_subcores=NS)

    @functools.partial(
        pl.kernel,
        out_type=jax.ShapeDtypeStruct((T, D), jnp.float32),
        mesh=mesh,
        scratch_types=[
            pltpu.VMEM((_PER_T,), jnp.int32),
            pltpu.VMEM((_PER_T,), jnp.int32),
            pltpu.VMEM((_PER_T, 16), jnp.float32),
            pltpu.VMEM((_PER_T, 16), jnp.float32),
            [pltpu.VMEM((_CHT, D), jnp.float32)] * 4,
            [pltpu.SemaphoreType.DMA] * 4,
        ],
    )
    def k(ys_hbm, p0_hbm, p1_hbm, w0_hbm, w1_hbm, out_hbm,
          i0_v, i1_v, wv0, wv1, bufs, sems):
        wid = lax.axis_index("s") * NC + lax.axis_index("c")
        base = wid * _PER_T
        pltpu.sync_copy(p0_hbm.at[pl.ds(base, _PER_T)], i0_v)
        pltpu.sync_copy(p1_hbm.at[pl.ds(base, _PER_T)], i1_v)
        pltpu.sync_copy(w0_hbm.at[pl.ds(base, _PER_T)], wv0)
        pltpu.sync_copy(w1_hbm.at[pl.ds(base, _PER_T)], wv1)
        cps = {}

        def fire(r):
            p = r % 2
            cps[(r, 0)] = pltpu.async_copy(
                ys_hbm.at[i0_v.at[pl.ds(r * _CHT, _CHT)]], bufs[2 * p], sems[2 * p])
            cps[(r, 1)] = pltpu.async_copy(
                ys_hbm.at[i1_v.at[pl.ds(r * _CHT, _CHT)]], bufs[2 * p + 1], sems[2 * p + 1])

        fire(0)
        fire(1)
        for r in range(_TROUNDS):
            p = r % 2
            cps[(r, 0)].wait()
            cps[(r, 1)].wait()
            b0, b1 = bufs[2 * p], bufs[2 * p + 1]

            def row_comb(row, _):
                a0 = wv0[r * _CHT + row, :]
                a1 = wv1[r * _CHT + row, :]
                for j in range(D // 16):
                    sl = (row, pl.ds(j * 16, 16))
                    b0[sl] = a0 * b0[sl] + a1 * b1[sl]
                return 0

            lax.fori_loop(0, _CHT, row_comb, 0)
            pltpu.sync_copy(b0, out_hbm.at[pl.ds(base + r * _CHT, _CHT)])
            if r + 2 < _TROUNDS:
                fire(r + 2)

    return k(ys, p0, p1, w0, w1)


def kernel(x, gating_output, gate_up_proj, down_proj):
    e1, e2, w1, w2, cnt = _router(gating_output)
    tok_sorted, slot_sorted, p0, p1, block_expert, valid, chg, nxt = (
        _routing_metadata(e1, e2, cnt))
    xs = _dispatch(x, tok_sorted, slot_sorted)
    ys = _ffn(xs, gate_up_proj, down_proj, block_expert, valid, chg, nxt)
    w0x = jnp.broadcast_to(w1, (T, 16))
    w1x = jnp.broadcast_to(w2, (T, 16))
    return _combine(ys, p0, p1, w0x, w1x)


# speedup vs baseline: 1.4707x; 1.4707x over previous
import functools

import jax
import jax.numpy as jnp
from jax import lax
from jax.experimental import pallas as pl
from jax.experimental.pallas import tpu as pltpu
from jax.experimental.pallas import tpu_sc as plsc

E = 8
K = 2
D = 1024
F = 2048
T = 2048
A = T * K
B = 256
NB = A // B + E
S = NB * B

NC = 2
NS = 16
NW = NC * NS


def _router_body(g_ref, e1_ref, e2_ref, w1_ref, w2_ref, cnt_ref):
    s = g_ref[...]
    ii = lax.broadcasted_iota(jnp.int32, (T, E), 1)
    m1 = jnp.max(s, axis=1, keepdims=True)
    i1 = jnp.min(jnp.where(s == m1, ii, E), axis=1, keepdims=True)
    s2 = jnp.where(ii == i1, -jnp.inf, s)
    m2 = jnp.max(s2, axis=1, keepdims=True)
    i2 = jnp.min(jnp.where(s2 == m2, ii, E), axis=1, keepdims=True)
    e1_ref[...] = i1
    e2_ref[...] = i2
    w1_ref[...] = jax.nn.sigmoid(m1 - m2)
    w2_ref[...] = jax.nn.sigmoid(m2 - m1)
    sel = (ii == i1).astype(jnp.int32) + (ii == i2).astype(jnp.int32)
    cnt_ref[...] = jnp.sum(sel, axis=0, keepdims=True)


def _router(gating):
    return pl.pallas_call(
        _router_body,
        out_shape=(
            jax.ShapeDtypeStruct((T, 1), jnp.int32),
            jax.ShapeDtypeStruct((T, 1), jnp.int32),
            jax.ShapeDtypeStruct((T, 1), jnp.float32),
            jax.ShapeDtypeStruct((T, 1), jnp.float32),
            jax.ShapeDtypeStruct((1, E), jnp.int32),
        ),
    )(gating)


def _routing_metadata(e1, e2, cnt):
    ea = jnp.concatenate([e1.reshape(T), e2.reshape(T)])
    ii = jnp.arange(A, dtype=jnp.int32)
    _, order = lax.sort((ea, ii), num_keys=1)
    tok_sorted = order % T
    g = cnt.reshape(E)
    c = (g + B - 1) // B
    cumg = jnp.cumsum(g).astype(jnp.int32)
    padc = (c * B - g).astype(jnp.int32)
    pad = jnp.sum(jnp.where(ii[:, None] >= cumg[None, :], padc[None, :], 0),
                  axis=1).astype(jnp.int32)
    slot_sorted = ii + pad
    _, inv_slot = lax.sort((order, slot_sorted), num_keys=1)
    p0 = inv_slot[:T]
    p1 = inv_slot[T:]
    block_expert = jnp.repeat(
        jnp.arange(E, dtype=jnp.int32), c, total_repeat_length=NB)
    nb_real = jnp.sum(c).astype(jnp.int32)
    bidx = jnp.arange(NB, dtype=jnp.int32)
    valid = (bidx < nb_real).astype(jnp.int32)
    chg_raw = jnp.concatenate(
        [jnp.ones(1, jnp.bool_), block_expert[1:] != block_expert[:-1]])
    chg = (chg_raw & (valid == 1)).astype(jnp.int32)
    cpos = jnp.where(chg == 1, bidx, NB - 1)
    nxtpos_incl = jnp.flip(lax.cummin(jnp.flip(cpos)))
    nxtpos = jnp.concatenate([nxtpos_incl[1:], jnp.full(1, NB - 1, jnp.int32)])
    nxt = block_expert[nxtpos]
    return tok_sorted, slot_sorted, p0, p1, block_expert, valid, chg, nxt


_RPW = A // NW
_CH = 32
_DROUNDS = _RPW // _CH


def _dispatch(x, tok_sorted, slot_sorted):
    mesh = plsc.VectorSubcoreMesh(
        core_axis_name="c", subcore_axis_name="s", num_cores=NC, num_subcores=NS)

    @functools.partial(
        pl.kernel,
        out_type=jax.ShapeDtypeStruct((S, D), jnp.float32),
        mesh=mesh,
        scratch_types=[
            [pltpu.VMEM((_CH,), jnp.int32)] * 2,
            [pltpu.VMEM((_CH,), jnp.int32)] * 2,
            [pltpu.VMEM((_CH, D), jnp.float32)] * 2,
            [pltpu.SemaphoreType.DMA] * 2,
            [pltpu.SemaphoreType.DMA] * 2,
        ],
    )
    def k(x_hbm, tok_hbm, slot_hbm, out_hbm, toks, slots, bufs, gsems, ssems):
        wid = lax.axis_index("s") * NC + lax.axis_index("c")
        base = wid * _RPW
        gcp = [None] * _DROUNDS
        scp = [None] * _DROUNDS

        def fire(r):
            p = r % 2
            pltpu.sync_copy(tok_hbm.at[pl.ds(base + r * _CH, _CH)], toks[p])
            pltpu.sync_copy(slot_hbm.at[pl.ds(base + r * _CH, _CH)], slots[p])
            gcp[r] = pltpu.async_copy(x_hbm.at[toks[p]], bufs[p], gsems[p])

        fire(0)
        for r in range(_DROUNDS):
            p = r % 2
            if r + 1 < _DROUNDS:
                fire(r + 1)
            gcp[r].wait()
            scp[r] = pltpu.async_copy(bufs[p], out_hbm.at[slots[p]], ssems[p])
            if r >= 1:
                scp[r - 1].wait()
        scp[_DROUNDS - 1].wait()

    return k(x, tok_sorted, slot_sorted)


def _ffn_body(be_ref, va_ref, chg_ref, nxt_ref, xs_ref, gup_ref, down_ref,
              ys_ref, gup_land, down_land, gup16, down16, gsem, dsem):
    b = pl.program_id(0)
    live = va_ref[b] != 0

    @pl.when(b == 0)
    def _():
        e0 = be_ref[0]
        pltpu.make_async_copy(gup_ref.at[e0], gup_land, gsem).start()
        pltpu.make_async_copy(down_ref.at[e0], down_land, dsem).start()

    @pl.when(chg_ref[b] == 1)
    def _():
        pltpu.make_async_copy(gup_ref.at[0], gup_land, gsem).wait()
        pltpu.make_async_copy(down_ref.at[0], down_land, dsem).wait()
        gup16[...] = gup_land[...].astype(jnp.bfloat16)
        down16[...] = down_land[...].astype(jnp.bfloat16)
        nxt = nxt_ref[b]

        @pl.when(nxt != be_ref[b])
        def _():
            pltpu.make_async_copy(gup_ref.at[nxt], gup_land, gsem).start()
            pltpu.make_async_copy(down_ref.at[nxt], down_land, dsem).start()

    @pl.when(live)
    def _():
        xb = xs_ref[...].astype(jnp.bfloat16)
        acc = lax.dot_general(xb, gup16[...], (((1,), (1,)), ((), ())),
                              preferred_element_type=jnp.float32)
        gte = acc[:, :F]
        up = acc[:, F:]
        h = (gte * jax.nn.sigmoid(gte) * up).astype(jnp.bfloat16)
        y = lax.dot_general(h, down16[...], (((1,), (1,)), ((), ())),
                            preferred_element_type=jnp.float32)
        ys_ref[...] = y


def _ffn(xs, gup, down, block_expert, valid, chg, nxt):
    grid_spec = pltpu.PrefetchScalarGridSpec(
        num_scalar_prefetch=4,
        grid=(NB,),
        in_specs=[
            pl.BlockSpec((B, D), lambda b, be, va, ch, nx: (b, 0)),
            pl.BlockSpec(memory_space=pltpu.MemorySpace.HBM),
            pl.BlockSpec(memory_space=pltpu.MemorySpace.HBM),
        ],
        out_specs=pl.BlockSpec((B, D), lambda b, be, va, ch, nx: (b, 0)),
        scratch_shapes=[
            pltpu.VMEM((2 * F, D), jnp.float32),
            pltpu.VMEM((D, F), jnp.float32),
            pltpu.VMEM((2 * F, D), jnp.bfloat16),
            pltpu.VMEM((D, F), jnp.bfloat16),
            pltpu.SemaphoreType.DMA,
            pltpu.SemaphoreType.DMA,
        ],
    )
    return pl.pallas_call(
        _ffn_body,
        grid_spec=grid_spec,
        out_shape=jax.ShapeDtypeStruct((S, D), jnp.float32),
    )(block_expert, valid, chg, nxt, xs, gup, down)


_PER_T = T // NW
_CHT = 16
_TROUNDS = _PER_T // _CHT


def _combine(ys, p0, p1, w0, w1):
    mesh = plsc.VectorSubcoreMesh(
        core_axis_name="c", subcore_axis_name="s", num_cores=NC, num_subcores=NS)

    @functools.partial(
        pl.kernel,
        out_type=jax.ShapeDtypeStruct((T, D), jnp.float32),
        mesh=mesh,
        scratch_types=[
            pltpu.VMEM((_PER_T,), jnp.int32),
            pltpu.VMEM((_PER_T,), jnp.int32),
            pltpu.VMEM((_PER_T, 16), jnp.float32),
            pltpu.VMEM((_PER_T, 16), jnp.float32),
            [pltpu.VMEM((_CHT, D), jnp.float32)] * 4,
            [pltpu.SemaphoreType.DMA] * 4,
        ],
    )
    def k(ys_hbm, p0_hbm, p1_hbm, w0_hbm, w1_hbm, out_hbm,
          i0_v, i1_v, wv0, wv1, bufs, sems):
        wid = lax.axis_index("s") * NC + lax.axis_index("c")
        base = wid * _PER_T
        pltpu.sync_copy(p0_hbm.at[pl.ds(base, _PER_T)], i0_v)
        pltpu.sync_copy(p1_hbm.at[pl.ds(base, _PER_T)], i1_v)
        pltpu.sync_copy(w0_hbm.at[pl.ds(base, _PER_T)], wv0)
        pltpu.sync_copy(w1_hbm.at[pl.ds(base, _PER_T)], wv1)
        cps = {}

        def fire(r):
            p = r % 2
            cps[(r, 0)] = pltpu.async_copy(
                ys_hbm.at[i0_v.at[pl.ds(r * _CHT, _CHT)]], bufs[2 * p], sems[2 * p])
            cps[(r, 1)] = pltpu.async_copy(
                ys_hbm.at[i1_v.at[pl.ds(r * _CHT, _CHT)]], bufs[2 * p + 1], sems[2 * p + 1])

        fire(0)
        fire(1)
        for r in range(_TROUNDS):
            p = r % 2
            cps[(r, 0)].wait()
            cps[(r, 1)].wait()
            b0, b1 = bufs[2 * p], bufs[2 * p + 1]

            def row_comb(row, _):
                a0 = wv0[r * _CHT + row, :]
                a1 = wv1[r * _CHT + row, :]
                for j in range(D // 16):
                    sl = (row, pl.ds(j * 16, 16))
                    b0[sl] = a0 * b0[sl] + a1 * b1[sl]
                return 0

            lax.fori_loop(0, _CHT, row_comb, 0)
            pltpu.sync_copy(b0, out_hbm.at[pl.ds(base + r * _CHT, _CHT)])
            if r + 2 < _TROUNDS:
                fire(r + 2)

    return k(ys, p0, p1, w0, w1)


def kernel(x, gating_output, gate_up_proj, down_proj):
    e1, e2, w1, w2, cnt = _router(gating_output)
    tok_sorted, slot_sorted, p0, p1, block_expert, valid, chg, nxt = (
        _routing_metadata(e1, e2, cnt))
    xs = _dispatch(x, tok_sorted, slot_sorted)
    ys = _ffn(xs, gate_up_proj, down_proj, block_expert, valid, chg, nxt)
    w0x = jnp.broadcast_to(w1, (T, 16))
    w1x = jnp.broadcast_to(w2, (T, 16))
    return _combine(ys, p0, p1, w0x, w1x)
